# Initial kernel scaffold; baseline (speedup 1.0000x reference)
import jax
import jax.numpy as jnp
from jax.experimental import pallas as pl


def kernel(x, params):
    raise NotImplementedError("write your pallas kernel here")


# speedup vs baseline: 3.3633x; 3.3633x over previous
import functools

import jax
import jax.numpy as jnp
import numpy as np
from jax.experimental import pallas as pl

EPS = 1e-5
_NEG = np.float32(-3.0e38)


def _bn(x):
    return x / jnp.sqrt(1.0 + EPS)


def _lrelu(x):
    return jax.nn.leaky_relu(x, negative_slope=0.2)


def _conv2d(w, x):
    return jnp.einsum('oi,bihw->bohw', w, x)


def _conv1d(w, x, b=None):
    y = jnp.einsum('oi,bin->bon', w, x)
    if b is not None:
        y = y + b[None, :, None]
    return y


def _knn_body(xt_ref, xc_ref, xx_ref, out_ref, *, K, N):
    rows = xt_ref[...].astype(jnp.bfloat16)
    dist = jax.lax.dot_general(
        rows, xc_ref[...].astype(jnp.bfloat16), (((1,), (0,)), ((), ())),
        preferred_element_type=jnp.float32)
    rank = 2.0 * dist - xx_ref[...]
    npad = rank.shape[1]
    iota = jax.lax.broadcasted_iota(jnp.int32, rank.shape, 1)
    d = jnp.where(iota < N, rank, _NEG)
    cols = []
    for t in range(K + 1):
        m = jnp.max(d, axis=1, keepdims=True)
        a = jnp.min(jnp.where(d == m, iota, npad), axis=1, keepdims=True)
        if t > 0:
            cols.append(a)
        d = jnp.where(iota == a, _NEG, d)
    out_ref[...] = jnp.concatenate(cols, axis=1)


def _knn_pallas(x, k):
    _, C, N = x.shape
    R = 256
    npad = ((N + R - 1) // R) * R
    cpad = ((C + 7) // 8) * 8
    xc = jnp.pad(x[0], ((0, cpad - C), (0, npad - N)))
    xt = xc.T
    xx = jnp.sum(x[0] * x[0], axis=0)
    xxp = jnp.pad(xx, (0, npad - N)).reshape(1, npad)
    out = pl.pallas_call(
        functools.partial(_knn_body, K=k, N=N),
        grid=(npad // R,),
        in_specs=[
            pl.BlockSpec((R, cpad), lambda i: (i, 0)),
            pl.BlockSpec((cpad, npad), lambda i: (0, 0)),
            pl.BlockSpec((1, npad), lambda i: (0, 0)),
        ],
        out_specs=pl.BlockSpec((R, k), lambda i: (i, 0)),
        out_shape=jax.ShapeDtypeStruct((npad, k), jnp.int32),
    )(xt, xc, xxp)
    return out[:N][None]


def _index_points(points, idx):
    return jax.vmap(lambda p, i: p[i])(points, idx)


def _get_graph_feature(coor, nor, k):
    idx = _knn_pallas(coor, k)
    coor_t = jnp.transpose(coor, (0, 2, 1))
    nor_t = jnp.transpose(nor, (0, 2, 1))
    cf = _index_points(coor_t, idx)
    nf = _index_points(nor_t, idx)
    cc = jnp.broadcast_to(coor_t[:, :, None, :], cf.shape)
    nn_ = jnp.broadcast_to(nor_t[:, :, None, :], nf.shape)
    coor_feature = jnp.transpose(jnp.concatenate([cf - cc, cc], axis=3), (0, 3, 1, 2))
    nor_feature = jnp.transpose(jnp.concatenate([nf - nn_, nn_], axis=3), (0, 3, 1, 2))
    return coor_feature, nor_feature, idx


def _graph_attention(w, idx, x, feature, K):
    B, C, N = x.shape
    xr = jnp.reshape(x, (B, N, C))
    feat = jnp.transpose(feature, (0, 2, 3, 1))
    neighbor = _index_points(xr, idx)
    centre = jnp.broadcast_to(xr[:, :, None, :], (B, N, K, C))
    delta_f = jnp.transpose(jnp.concatenate([centre - neighbor, neighbor], axis=3), (0, 3, 2, 1))
    e = _lrelu(_bn(_conv2d(w, delta_f)))
    e = jnp.transpose(e, (0, 3, 2, 1))
    attention = jax.nn.softmax(e, axis=2)
    return jnp.transpose(jnp.sum(attention * feat, axis=2), (0, 2, 1))


def _nonlocal_block(p, idx, x, feature):
    B, C, N = x.shape
    xr = jnp.reshape(x, (B, N, C))
    neighbor = _index_points(xr, idx)
    centre = jnp.transpose(xr[:, :, None, :], (0, 3, 2, 1))
    centre = _conv2d(p['theta_w'], centre) + p['theta_b'][None, :, None, None]
    theta_x = jnp.transpose(centre, (0, 3, 2, 1))
    phi = jnp.transpose(neighbor, (0, 3, 2, 1))
    phi = _conv2d(p['theta_w'], phi) + p['theta_b'][None, :, None, None]
    phi_x = jnp.transpose(phi, (0, 3, 1, 2))
    mid = jnp.matmul(theta_x, phi_x)
    coeff = jax.nn.softmax(mid, axis=3)
    feat = _conv2d(p['g_w'], feature) + p['g_b'][None, :, None, None]
    g_x = jnp.transpose(feat, (0, 2, 3, 1))
    output = jnp.matmul(coeff, g_x)
    output = jnp.transpose(jnp.sum(output, axis=2), (0, 2, 1))
    return _bn(_conv1d(p['W_w'], output, p['W_b']))


def kernel(x, params):
    p = params
    coor = x[:, :3, :]
    nor = x[:, 3:, :]
    cf1, nf1, idx1 = _get_graph_feature(coor, nor, 16)
    c1 = _lrelu(_bn(_conv2d(p['conv1_c_w'], cf1)))
    n1 = _lrelu(_bn(_conv2d(p['conv1_n_w'], nf1)))
    coor1 = _graph_attention(p['att1_w'], idx1, coor, c1, 16)
    nor1 = _nonlocal_block(p['nlb1'], idx1, nor, n1)
    cf2, nf2, idx2 = _get_graph_feature(coor1, nor1, 16)
    c2 = _lrelu(_bn(_conv2d(p['conv2_c_w'], cf2)))
    n2 = _lrelu(_bn(_conv2d(p['conv2_n_w'], nf2)))
    coor2 = _graph_attention(p['att2_w'], idx2, coor1, c2, 16)
    nor2 = _nonlocal_block(p['nlb2'], idx2, nor1, n2)
    cf3, nf3, idx3 = _get_graph_feature(coor2, nor2, 32)
    c3 = _lrelu(_bn(_conv2d(p['conv3_c_w'], cf3)))
    n3 = _lrelu(_bn(_conv2d(p['conv3_n_w'], nf3)))
    coor3 = _graph_attention(p['att3_w'], idx3, coor2, c3, 32)
    nor3 = _nonlocal_block(p['nlb3'], idx3, nor2, n3)
    coor_cat = jnp.concatenate([coor1, coor2, coor3], axis=1)
    nor_cat = jnp.concatenate([nor1, nor2, nor3], axis=1)
    cfeat = _lrelu(_bn(_conv1d(p['conv5_c_w'], coor_cat)))
    nfeat = _lrelu(_bn(_conv1d(p['conv5_n_w'], nor_cat)))
    feat = jnp.concatenate([cfeat, nfeat], axis=1)
    feat = _lrelu(_bn(_conv1d(p['conv6_w'], feat)))
    feat = _lrelu(_bn(_conv1d(p['conv7_w'], feat)))
    score = _conv1d(p['pred_w'], feat, p['pred_b'])
    return jnp.transpose(score, (0, 2, 1))


# speedup vs baseline: 12.8452x; 3.8193x over previous
import functools

import jax
import jax.numpy as jnp
import numpy as np
from jax import lax
from jax.experimental import pallas as pl
from jax.experimental.pallas import tpu as pltpu
from jax.experimental.pallas import tpu_sc as plsc

EPS = 1e-5
_NEG = np.float32(-3.0e38)


def _bn(x):
    return x / jnp.sqrt(1.0 + EPS)


def _lrelu(x):
    return jax.nn.leaky_relu(x, negative_slope=0.2)


def _conv2d(w, x):
    return jnp.einsum('oi,bihw->bohw', w, x)


def _conv1d(w, x, b=None):
    y = jnp.einsum('oi,bin->bon', w, x)
    if b is not None:
        y = y + b[None, :, None]
    return y


def _knn_body(xt_ref, xc_ref, xx_ref, out_ref, *, K, N):
    rows = xt_ref[...].astype(jnp.bfloat16)
    dist = jax.lax.dot_general(
        rows, xc_ref[...].astype(jnp.bfloat16), (((1,), (0,)), ((), ())),
        preferred_element_type=jnp.float32)
    rank = 2.0 * dist - xx_ref[...]
    npad = rank.shape[1]
    iota = jax.lax.broadcasted_iota(jnp.int32, rank.shape, 1)
    d = jnp.where(iota < N, rank, _NEG)
    cols = []
    for t in range(K + 1):
        m = jnp.max(d, axis=1, keepdims=True)
        a = jnp.min(jnp.where(d == m, iota, npad), axis=1, keepdims=True)
        if t > 0:
            cols.append(a)
        d = jnp.where(iota == a, _NEG, d)
    out_ref[...] = jnp.concatenate(cols, axis=1)


def _knn_pallas(x, k):
    _, C, N = x.shape
    R = 256
    npad = ((N + R - 1) // R) * R
    cpad = ((C + 7) // 8) * 8
    xc = jnp.pad(x[0], ((0, cpad - C), (0, npad - N)))
    xt = xc.T
    xx = jnp.sum(x[0] * x[0], axis=0)
    xxp = jnp.pad(xx, (0, npad - N)).reshape(1, npad)
    out = pl.pallas_call(
        functools.partial(_knn_body, K=k, N=N),
        grid=(npad // R,),
        in_specs=[
            pl.BlockSpec((R, cpad), lambda i: (i, 0)),
            pl.BlockSpec((cpad, npad), lambda i: (0, 0)),
            pl.BlockSpec((1, npad), lambda i: (0, 0)),
        ],
        out_specs=pl.BlockSpec((R, k), lambda i: (i, 0)),
        out_shape=jax.ShapeDtypeStruct((npad, k), jnp.int32),
    )(xt, xc, xxp)
    return out[:N][None]


def _sc_gather(table, idx):
    M = idx.shape[0]
    D = table.shape[1]
    NW = 32
    per_w = M // NW
    ch = None
    for cand in (1000, 800, 400, 200, 40, 8):
        if per_w % cand == 0 and cand * D * 4 <= 420_000:
            ch = cand
            break
    mesh = plsc.VectorSubcoreMesh(core_axis_name="c", subcore_axis_name="s")

    @functools.partial(
        pl.kernel, mesh=mesh,
        out_type=jax.ShapeDtypeStruct((M, D), jnp.float32),
        scratch_types=[
            pltpu.VMEM((ch,), jnp.int32),
            pltpu.VMEM((ch, D), jnp.float32),
            pltpu.SemaphoreType.DMA,
        ],
    )
    def gk(tab_hbm, idx_hbm, out_hbm, idx_v, rows_v, sem):
        wid = lax.axis_index("s") * 2 + lax.axis_index("c")
        base = wid * per_w

        def body(j, carry):
            off = base + j * ch
            pltpu.sync_copy(idx_hbm.at[pl.ds(off, ch)], idx_v)
            pltpu.async_copy(tab_hbm.at[idx_v], rows_v, sem).wait()
            pltpu.sync_copy(rows_v, out_hbm.at[pl.ds(off, ch)])
            return carry

        lax.fori_loop(0, per_w // ch, body, 0)

    return gk(table, idx)


def _gather4(coor, nor, idx):
    _, C, N = coor.shape
    K = idx.shape[2]
    tabs = jnp.concatenate(
        [coor[0].T, nor[0].T, coor[0].reshape(N, C), nor[0].reshape(N, C)], axis=1)
    D = 4 * C
    dpad = ((D + 127) // 128) * 128
    if dpad != D:
        tabs = jnp.pad(tabs, ((0, 0), (0, dpad - D)))
    g = _sc_gather(tabs, idx[0].reshape(-1))
    g = g.reshape(N, K, dpad)
    return (g[None, :, :, 0:C], g[None, :, :, C:2 * C],
            g[None, :, :, 2 * C:3 * C], g[None, :, :, 3 * C:4 * C])


def _graph_feature_from(coor, nor, cf, nf):
    coor_t = jnp.transpose(coor, (0, 2, 1))
    nor_t = jnp.transpose(nor, (0, 2, 1))
    cc = jnp.broadcast_to(coor_t[:, :, None, :], cf.shape)
    nn_ = jnp.broadcast_to(nor_t[:, :, None, :], nf.shape)
    coor_feature = jnp.transpose(jnp.concatenate([cf - cc, cc], axis=3), (0, 3, 1, 2))
    nor_feature = jnp.transpose(jnp.concatenate([nf - nn_, nn_], axis=3), (0, 3, 1, 2))
    return coor_feature, nor_feature


def _graph_attention(w, neighbor, x, feature, K):
    B, C, N = x.shape
    xr = jnp.reshape(x, (B, N, C))
    feat = jnp.transpose(feature, (0, 2, 3, 1))
    centre = jnp.broadcast_to(xr[:, :, None, :], (B, N, K, C))
    delta_f = jnp.transpose(jnp.concatenate([centre - neighbor, neighbor], axis=3), (0, 3, 2, 1))
    e = _lrelu(_bn(_conv2d(w, delta_f)))
    e = jnp.transpose(e, (0, 3, 2, 1))
    attention = jax.nn.softmax(e, axis=2)
    return jnp.transpose(jnp.sum(attention * feat, axis=2), (0, 2, 1))


def _nonlocal_block(p, neighbor, x, feature):
    B, C, N = x.shape
    xr = jnp.reshape(x, (B, N, C))
    centre = jnp.transpose(xr[:, :, None, :], (0, 3, 2, 1))
    centre = _conv2d(p['theta_w'], centre) + p['theta_b'][None, :, None, None]
    theta_x = jnp.transpose(centre, (0, 3, 2, 1))
    phi = jnp.transpose(neighbor, (0, 3, 2, 1))
    phi = _conv2d(p['theta_w'], phi) + p['theta_b'][None, :, None, None]
    phi_x = jnp.transpose(phi, (0, 3, 1, 2))
    mid = jnp.matmul(theta_x, phi_x)
    coeff = jax.nn.softmax(mid, axis=3)
    feat = _conv2d(p['g_w'], feature) + p['g_b'][None, :, None, None]
    g_x = jnp.transpose(feat, (0, 2, 3, 1))
    output = jnp.matmul(coeff, g_x)
    output = jnp.transpose(jnp.sum(output, axis=2), (0, 2, 1))
    return _bn(_conv1d(p['W_w'], output, p['W_b']))


def _layer(coor, nor, k, conv_c_w, conv_n_w, att_w, nlb):
    idx = _knn_pallas(coor, k)
    cf, nf, xg, ng = _gather4(coor, nor, idx)
    coor_feature, nor_feature = _graph_feature_from(coor, nor, cf, nf)
    c = _lrelu(_bn(_conv2d(conv_c_w, coor_feature)))
    n = _lrelu(_bn(_conv2d(conv_n_w, nor_feature)))
    coor_out = _graph_attention(att_w, xg, coor, c, k)
    nor_out = _nonlocal_block(nlb, ng, nor, n)
    return coor_out, nor_out


def kernel(x, params):
    p = params
    coor = x[:, :3, :]
    nor = x[:, 3:, :]
    coor1, nor1 = _layer(coor, nor, 16, p['conv1_c_w'], p['conv1_n_w'], p['att1_w'], p['nlb1'])
    coor2, nor2 = _layer(coor1, nor1, 16, p['conv2_c_w'], p['conv2_n_w'], p['att2_w'], p['nlb2'])
    coor3, nor3 = _layer(coor2, nor2, 32, p['conv3_c_w'], p['conv3_n_w'], p['att3_w'], p['nlb3'])
    coor_cat = jnp.concatenate([coor1, coor2, coor3], axis=1)
    nor_cat = jnp.concatenate([nor1, nor2, nor3], axis=1)
    cfeat = _lrelu(_bn(_conv1d(p['conv5_c_w'], coor_cat)))
    nfeat = _lrelu(_bn(_conv1d(p['conv5_n_w'], nor_cat)))
    feat = jnp.concatenate([cfeat, nfeat], axis=1)
    feat = _lrelu(_bn(_conv1d(p['conv6_w'], feat)))
    feat = _lrelu(_bn(_conv1d(p['conv7_w'], feat)))
    score = _conv1d(p['pred_w'], feat, p['pred_b'])
    return jnp.transpose(score, (0, 2, 1))


# speedup vs baseline: 15.1180x; 1.1769x over previous
import functools

import jax
import jax.numpy as jnp
import numpy as np
from jax import lax
from jax.experimental import pallas as pl
from jax.experimental.pallas import tpu as pltpu
from jax.experimental.pallas import tpu_sc as plsc

EPS = 1e-5
_NEG = np.float32(-3.0e38)


def _bn(x):
    return x / jnp.sqrt(1.0 + EPS)


def _lrelu(x):
    return jax.nn.leaky_relu(x, negative_slope=0.2)


def _conv2d(w, x):
    return jnp.einsum('oi,bihw->bohw', w, x)


def _conv1d(w, x, b=None):
    y = jnp.einsum('oi,bin->bon', w, x)
    if b is not None:
        y = y + b[None, :, None]
    return y


def _knn_body(xt_ref, xc_ref, xx_ref, out_ref, *, K, N):
    rows = xt_ref[...].astype(jnp.bfloat16)
    dist = jax.lax.dot_general(
        rows, xc_ref[...].astype(jnp.bfloat16), (((1,), (0,)), ((), ())),
        preferred_element_type=jnp.float32)
    rank = 2.0 * dist - xx_ref[...]
    npad = rank.shape[1]
    iota = jax.lax.broadcasted_iota(jnp.int32, rank.shape, 1)
    d = jnp.where(iota < N, rank, _NEG)
    cols = []
    for t in range(K + 1):
        m = jnp.max(d, axis=1, keepdims=True)
        a = jnp.min(jnp.where(d == m, iota, npad), axis=1, keepdims=True)
        if t > 0:
            cols.append(a)
        d = jnp.where(iota == a, _NEG, d)
    out_ref[...] = jnp.concatenate(cols, axis=1)


def _knn_pallas(x, k):
    _, C, N = x.shape
    R = 256
    npad = ((N + R - 1) // R) * R
    cpad = ((C + 7) // 8) * 8
    xc = jnp.pad(x[0], ((0, cpad - C), (0, npad - N)))
    xt = xc.T
    xx = jnp.sum(x[0] * x[0], axis=0)
    xxp = jnp.pad(xx, (0, npad - N)).reshape(1, npad)
    out = pl.pallas_call(
        functools.partial(_knn_body, K=k, N=N),
        grid=(npad // R,),
        in_specs=[
            pl.BlockSpec((R, cpad), lambda i: (i, 0)),
            pl.BlockSpec((cpad, npad), lambda i: (0, 0)),
            pl.BlockSpec((1, npad), lambda i: (0, 0)),
        ],
        out_specs=pl.BlockSpec((R, k), lambda i: (i, 0)),
        out_shape=jax.ShapeDtypeStruct((npad, k), jnp.int32),
    )(xt, xc, xxp)
    return out[:N][None]


def _sc_gather(table, idx):
    M = idx.shape[0]
    D = table.shape[1]
    NW = 32
    per_w = M // NW
    ch = None
    for cand in (1000, 800, 400, 200, 40, 8):
        if per_w % cand == 0 and cand * D * 4 <= 420_000:
            ch = cand
            break
    mesh = plsc.VectorSubcoreMesh(core_axis_name="c", subcore_axis_name="s")

    @functools.partial(
        pl.kernel, mesh=mesh,
        out_type=jax.ShapeDtypeStruct((M, D), jnp.float32),
        scratch_types=[
            pltpu.VMEM((ch,), jnp.int32),
            pltpu.VMEM((ch, D), jnp.float32),
            pltpu.SemaphoreType.DMA,
        ],
    )
    def gk(tab_hbm, idx_hbm, out_hbm, idx_v, rows_v, sem):
        wid = lax.axis_index("s") * 2 + lax.axis_index("c")
        base = wid * per_w

        def body(j, carry):
            off = base + j * ch
            pltpu.sync_copy(idx_hbm.at[pl.ds(off, ch)], idx_v)
            pltpu.async_copy(tab_hbm.at[idx_v], rows_v, sem).wait()
            pltpu.sync_copy(rows_v, out_hbm.at[pl.ds(off, ch)])
            return carry

        lax.fori_loop(0, per_w // ch, body, 0)

    return gk(table, idx)


def _build_tabs(coor, nor):
    _, C, N = coor.shape
    slot = max(32, C)
    cols = []
    for a in (coor[0].T, nor[0].T, coor[0].reshape(N, C), nor[0].reshape(N, C)):
        cols.append(a if C == slot else jnp.pad(a, ((0, 0), (0, slot - C))))
    tabs = jnp.concatenate(cols, axis=1)
    dpad = ((4 * slot + 127) // 128) * 128
    if dpad != 4 * slot:
        tabs = jnp.pad(tabs, ((0, 0), (0, dpad - 4 * slot)))
    return tabs, slot


def _edge_body(tabs_ref, g_ref, wc_ref, wn_ref, wa_ref, th_ref, thb_ref,
               gw_ref, gb_ref, ww_ref, wb_ref, cout_ref, nout_ref,
               *, C, C2, K, slot):
    tb = tabs_ref[...]
    ct = tb[:, 0:C]
    nt = tb[:, slot:slot + C]
    xr = tb[:, 2 * slot:2 * slot + C]
    nr = tb[:, 3 * slot:3 * slot + C]
    g = g_ref[...]
    cf = g[:, :, 0:C]
    nf = g[:, :, slot:slot + C]
    xg = g[:, :, 2 * slot:2 * slot + C]
    ng = g[:, :, 3 * slot:3 * slot + C]
    R = ct.shape[0]
    E = K * R

    def rep(a):
        return jnp.broadcast_to(a[None], (K,) + a.shape)

    def mm(x, w):
        return jax.lax.dot_general(
            x.astype(jnp.bfloat16), w.astype(jnp.bfloat16),
            (((1,), (0,)), ((), ())), preferred_element_type=jnp.float32)

    def act(y):
        return _lrelu(_bn(y))

    ctr = rep(ct)
    ntr = rep(nt)
    c3 = act(mm(jnp.concatenate([cf - ctr, ctr], axis=2).reshape(E, 2 * C),
                wc_ref[...])).reshape(K, R, -1)
    n = act(mm(jnp.concatenate([nf - ntr, ntr], axis=2).reshape(E, 2 * C),
               wn_ref[...]))
    xrr = rep(xr)
    e3 = act(mm(jnp.concatenate([xrr - xg, xg], axis=2).reshape(E, 2 * C),
                wa_ref[...])).reshape(K, R, -1)
    emax = jnp.max(e3, axis=0, keepdims=True)
    p = jnp.exp(e3 - emax)
    att = p / jnp.sum(p, axis=0, keepdims=True)
    cout_ref[...] = jnp.sum(att * c3, axis=0)

    tcn = mm(nr, th_ref[...]) + thb_ref[...]
    tnn = (mm(ng.reshape(E, C), th_ref[...]) + thb_ref[...]).reshape(K, R, C2)
    mid = jnp.sum(rep(tcn) * tnn, axis=2, keepdims=True)
    mmax = jnp.max(mid, axis=0, keepdims=True)
    pp = jnp.exp(mid - mmax)
    coeff = pp / jnp.sum(pp, axis=0, keepdims=True)
    g1 = (mm(n, gw_ref[...]) + gb_ref[...]).reshape(K, R, -1)
    out = jnp.sum(coeff * g1, axis=0)
    nout_ref[...] = _bn(mm(out, ww_ref[...]) + wb_ref[...])


def _edge_pallas(tabs, gfl, k, conv_c_w, conv_n_w, att_w, nlb, C, slot):
    N, dpad = tabs.shape
    R = 200
    o1 = conv_c_w.shape[0]
    o2 = att_w.shape[0]
    C2 = nlb['theta_w'].shape[0]
    g3 = gfl.reshape(k, N, dpad)
    wc = conv_c_w.T
    wn = conv_n_w.T
    wa = att_w.T
    th = nlb['theta_w'].T
    thb = nlb['theta_b'].reshape(1, C2)
    gw = nlb['g_w'].T
    gb = nlb['g_b'].reshape(1, -1)
    ww = nlb['W_w'].T
    wb = nlb['W_b'].reshape(1, -1)
    couto, nouto = pl.pallas_call(
        functools.partial(_edge_body, C=C, C2=C2, K=k, slot=slot),
        grid=(N // R,),
        in_specs=[
            pl.BlockSpec((R, dpad), lambda i: (i, 0)),
            pl.BlockSpec((k, R, dpad), lambda i: (0, i, 0)),
            pl.BlockSpec(wc.shape, lambda i: (0, 0)),
            pl.BlockSpec(wn.shape, lambda i: (0, 0)),
            pl.BlockSpec(wa.shape, lambda i: (0, 0)),
            pl.BlockSpec(th.shape, lambda i: (0, 0)),
            pl.BlockSpec(thb.shape, lambda i: (0, 0)),
            pl.BlockSpec(gw.shape, lambda i: (0, 0)),
            pl.BlockSpec(gb.shape, lambda i: (0, 0)),
            pl.BlockSpec(ww.shape, lambda i: (0, 0)),
            pl.BlockSpec(wb.shape, lambda i: (0, 0)),
        ],
        out_specs=[
            pl.BlockSpec((R, o2), lambda i: (i, 0)),
            pl.BlockSpec((R, ww.shape[1]), lambda i: (i, 0)),
        ],
        out_shape=[
            jax.ShapeDtypeStruct((N, o2), jnp.float32),
            jax.ShapeDtypeStruct((N, ww.shape[1]), jnp.float32),
        ],
    )(tabs, g3, wc, wn, wa, th, thb, gw, gb, ww, wb)
    return couto, nouto


def _head_body(cc_ref, nc_ref, w5c_ref, w5n_ref, w6_ref, w7_ref, wp_ref, bp_ref, out_ref):
    def mm(w, x):
        return jax.lax.dot_general(
            w.astype(jnp.bfloat16), x.astype(jnp.bfloat16),
            (((1,), (0,)), ((), ())), preferred_element_type=jnp.float32)

    def act(y):
        return _lrelu(_bn(y))

    cfeat = act(mm(w5c_ref[...], cc_ref[...]))
    nfeat = act(mm(w5n_ref[...], nc_ref[...]))
    feat = jnp.concatenate([cfeat, nfeat], axis=0)
    feat = act(mm(w6_ref[...], feat))
    feat = act(mm(w7_ref[...], feat))
    out_ref[...] = mm(wp_ref[...], feat) + bp_ref[...]


def _head_pallas(coor_cat, nor_cat, p):
    _, cin, N = coor_cat.shape
    T = 1024
    npad = ((N + T - 1) // T) * T
    cc = jnp.pad(coor_cat[0], ((0, 0), (0, npad - N)))
    nc = jnp.pad(nor_cat[0], ((0, 0), (0, npad - N)))
    wp = jnp.pad(p['pred_w'], ((0, 2), (0, 0)))
    bp = jnp.pad(p['pred_b'], (0, 2)).reshape(16, 1)
    out = pl.pallas_call(
        _head_body,
        grid=(npad // T,),
        in_specs=[
            pl.BlockSpec((cin, T), lambda i: (0, i)),
            pl.BlockSpec((cin, T), lambda i: (0, i)),
            pl.BlockSpec(p['conv5_c_w'].shape, lambda i: (0, 0)),
            pl.BlockSpec(p['conv5_n_w'].shape, lambda i: (0, 0)),
            pl.BlockSpec(p['conv6_w'].shape, lambda i: (0, 0)),
            pl.BlockSpec(p['conv7_w'].shape, lambda i: (0, 0)),
            pl.BlockSpec((16, 256), lambda i: (0, 0)),
            pl.BlockSpec((16, 1), lambda i: (0, 0)),
        ],
        out_specs=pl.BlockSpec((16, T), lambda i: (0, i)),
        out_shape=jax.ShapeDtypeStruct((16, npad), jnp.float32),
    )(cc, nc, p['conv5_c_w'], p['conv5_n_w'], p['conv6_w'], p['conv7_w'], wp, bp)
    return jnp.transpose(out[:14, :N])[None]


def _layer(coor, nor, k, conv_c_w, conv_n_w, att_w, nlb):
    C = coor.shape[1]
    idx = _knn_pallas(coor, k)
    tabs, slot = _build_tabs(coor, nor)
    gfl = _sc_gather(tabs, idx[0].T.reshape(-1))
    couto, nouto = _edge_pallas(tabs, gfl, k, conv_c_w, conv_n_w, att_w, nlb, C, slot)
    return couto.T[None], nouto.T[None]


def kernel(x, params):
    p = params
    coor = x[:, :3, :]
    nor = x[:, 3:, :]
    coor1, nor1 = _layer(coor, nor, 16, p['conv1_c_w'], p['conv1_n_w'], p['att1_w'], p['nlb1'])
    coor2, nor2 = _layer(coor1, nor1, 16, p['conv2_c_w'], p['conv2_n_w'], p['att2_w'], p['nlb2'])
    coor3, nor3 = _layer(coor2, nor2, 32, p['conv3_c_w'], p['conv3_n_w'], p['att3_w'], p['nlb3'])
    coor_cat = jnp.concatenate([coor1, coor2, coor3], axis=1)
    nor_cat = jnp.concatenate([nor1, nor2, nor3], axis=1)
    return _head_pallas(coor_cat, nor_cat, p)
